# Initial kernel scaffold; baseline (speedup 1.0000x reference)
import jax
import jax.numpy as jnp
from jax.experimental import pallas as pl


def kernel(x, edge_distance, node_counts, combine_sets, plane_wave, W_emb, W1v, b1v, W2vg, b2vg, W2v, b2v, Wg, bg, Wm, bm, Wp1, bp1, Wp2, bp2, Wpsi, bpsi, Wlr1, blr1, Wlr2, blr2, edge_sources, edge_targets, graph_indices):
    raise NotImplementedError("write your pallas kernel here")


# speedup vs baseline: 1.4883x; 1.4883x over previous
import functools

import jax
import jax.numpy as jnp
from jax import lax
from jax.experimental import pallas as pl
from jax.experimental.pallas import tpu as pltpu
from jax.experimental.pallas import tpu_sc as plsc

N = 10000
E = 320000
D_IN = 128
H = 64
NG = 128
CUTOFF = 8.0

HT = 128
_NC = 2
_NS = 16
_NW = _NC * _NS
_ROW = 128
_GRP = 8
_HALF = _GRP // 2
_CHUNK = _ROW * _HALF
E_PAD = 327680
_ROWS_PER_W = (E_PAD // _ROW) // _NW
_GRPS_PER_W = _ROWS_PER_W // _GRP
_EDGES_PER_W = E_PAD // _NW
N_PAD = 10240
_NPT = N_PAD // _NS
_ZCH = 128

_f32 = jnp.float32


def _sigmoid(v):
    return 1.0 / (1.0 + jnp.exp(-v))


def _elu(v):
    return jnp.where(v > 0, v, jnp.exp(jnp.minimum(v, 0.0)) - 1.0)


_MESH = None


def _mesh():
    global _MESH
    if _MESH is None:
        _MESH = plsc.VectorSubcoreMesh(core_axis_name="c", subcore_axis_name="s")
    return _MESH


def _gather_body(h_hbm, src_hbm, dst_hbm, ni_hbm, nj_hbm, idx_v, rows_v, sem):
    wid = lax.axis_index("s") * _NC + lax.axis_index("c")

    def run(ihbm, ohbm):
        def group(g, carry):
            row0 = wid * _ROWS_PER_W + g * _GRP
            pltpu.sync_copy(ihbm.at[pl.ds(row0, _GRP), :], idx_v)
            for half in range(2):
                descs = [
                    pltpu.async_copy(
                        h_hbm.at[idx_v.at[half * _HALF + j]],
                        rows_v.at[pl.ds(j * _ROW, _ROW), :],
                        sem,
                    )
                    for j in range(_HALF)
                ]
                for d in descs:
                    d.wait()
                e0 = wid * _EDGES_PER_W + g * (_GRP * _ROW) + half * _CHUNK
                pltpu.sync_copy(rows_v, ohbm.at[pl.ds(e0, _CHUNK), :])
            return carry

        lax.fori_loop(0, _GRPS_PER_W, group, 0)

    run(src_hbm, ni_hbm)
    run(dst_hbm, nj_hbm)


def _sc_gather(h, src2d, dst2d):
    f = functools.partial(
        pl.kernel,
        out_type=(
            jax.ShapeDtypeStruct((E_PAD, HT), _f32),
            jax.ShapeDtypeStruct((E_PAD, HT), _f32),
        ),
        mesh=_mesh(),
        scratch_types=[
            pltpu.VMEM((_GRP, _ROW), jnp.int32),
            pltpu.VMEM((_CHUNK, HT), _f32),
            pltpu.SemaphoreType.DMA,
        ],
    )(_gather_body)
    return f(h, src2d, dst2d)


def _scatter_body(z_hbm, idx_hbm, zero_hbm, out_hbm, table_sh, idx_v, z_v):
    cid = lax.axis_index("c")
    sid = lax.axis_index("s")
    wid = sid * _NC + cid

    pltpu.sync_copy(zero_hbm, z_v)
    for k in range(_NPT // _ZCH):
        pltpu.sync_copy(z_v, table_sh.at[pl.ds(sid * _NPT + k * _ZCH, _ZCH), :])
    plsc.subcore_barrier()

    def group(g, carry):
        row0 = wid * _ROWS_PER_W + g * _GRP
        pltpu.sync_copy(idx_hbm.at[pl.ds(row0, _GRP), :], idx_v)
        for j in range(_GRP):
            e0 = wid * _EDGES_PER_W + g * (_GRP * _ROW) + j * _ROW
            pltpu.sync_copy(z_hbm.at[pl.ds(e0, _ROW), :], z_v)
            pltpu.sync_copy(z_v, table_sh.at[idx_v.at[j]], add=True)
        return carry

    lax.fori_loop(0, _GRPS_PER_W, group, 0)
    plsc.subcore_barrier()

    for k in range(_NPT // _ZCH):
        r0 = sid * _NPT + k * _ZCH
        pltpu.sync_copy(table_sh.at[pl.ds(r0, _ZCH), :], z_v)
        pltpu.sync_copy(z_v, out_hbm.at[cid, pl.ds(r0, _ZCH), :])


def _sc_scatter(z, idx2d, zeros_tile):
    f = functools.partial(
        pl.kernel,
        out_type=jax.ShapeDtypeStruct((_NC, N_PAD, HT), _f32),
        mesh=_mesh(),
        scratch_types=[
            pltpu.VMEM_SHARED((N_PAD, HT), _f32),
            pltpu.VMEM((_GRP, _ROW), jnp.int32),
            pltpu.VMEM((_ZCH, HT), _f32),
        ],
    )(_scatter_body)
    return f(z, idx2d, zeros_tile)


_TN = 2000
_TE = 2048


def _embed_body(x_ref, w_ref, h_ref):
    hv = _sigmoid(jnp.dot(x_ref[...], w_ref[...], preferred_element_type=_f32))
    h_ref[...] = jnp.concatenate([hv, jnp.zeros((_TN, HT - H), _f32)], axis=1)


def _tc_embed(x, w_embT):
    return pl.pallas_call(
        _embed_body,
        grid=(N // _TN,),
        in_specs=[
            pl.BlockSpec((_TN, D_IN), lambda t: (t, 0)),
            pl.BlockSpec((D_IN, H), lambda t: (0, 0)),
        ],
        out_specs=pl.BlockSpec((_TN, HT), lambda t: (t, 0)),
        out_shape=jax.ShapeDtypeStruct((N, HT), _f32),
    )(x, w_embT)


def _edge_body(ni_ref, nj_ref, r_ref, cs_ref, pw_ref, ga1, ga2, ga3, bgm,
               w1t, b1, wgt, bgv, w2t, b2, z_ref):
    t = pl.program_id(0)
    rv = r_ref[...]
    inv = 1.0 / rv
    ni = ni_ref[:, :H]
    nj = nj_ref[:, :H]
    d = (ni - nj) * inv
    gm = (
        jnp.dot(ni, ga1[...], preferred_element_type=_f32)
        + jnp.dot(nj, ga2[...], preferred_element_type=_f32)
        + jnp.dot(d, ga3[...], preferred_element_type=_f32)
        + bgm[...]
    )
    g = _sigmoid(gm[:, :H])
    m = _elu(gm[:, H:])
    pw = pw_ref[...]
    z1 = jnp.dot(cs_ref[...], w1t[...], preferred_element_type=_f32) + b1[...]
    gt = _sigmoid(jnp.dot(pw, wgt[...], preferred_element_type=_f32) + bgv[...])
    z2 = jnp.dot(pw * gt, w2t[...], preferred_element_type=_f32) + b2[...]
    ids = t * _TE + lax.broadcasted_iota(jnp.int32, (_TE, 1), 0)
    keep = jnp.logical_and(ids < E, rv < CUTOFF)
    zv = jnp.where(keep, g * m * (z1 + z2), 0.0)
    z_ref[...] = jnp.concatenate([zv, jnp.zeros((_TE, HT - H), _f32)], axis=1)


def _tc_edge(ni, nj, rij, cs, pw, ga1, ga2, ga3, bgm, w1t, b1, wgt, bgv, w2t, b2):
    full = lambda a, b: pl.BlockSpec((a, b), lambda t: (0, 0))
    clamp = lambda t: (jnp.minimum(t, E // _TE), 0)
    return pl.pallas_call(
        _edge_body,
        grid=(E_PAD // _TE,),
        in_specs=[
            pl.BlockSpec((_TE, HT), lambda t: (t, 0)),
            pl.BlockSpec((_TE, HT), lambda t: (t, 0)),
            pl.BlockSpec((_TE, 1), clamp),
            pl.BlockSpec((_TE, H), clamp),
            pl.BlockSpec((_TE, H), clamp),
            full(H, 2 * H), full(H, 2 * H), full(H, 2 * H), full(1, 2 * H),
            full(H, H), full(1, H),
            full(H, H), full(1, H),
            full(H, H), full(1, H),
        ],
        out_specs=pl.BlockSpec((_TE, HT), lambda t: (t, 0)),
        out_shape=jax.ShapeDtypeStruct((E_PAD, HT), _f32),
    )(ni, nj, rij, cs, pw, ga1, ga2, ga3, bgm, w1t, b1, wgt, bgv, w2t, b2)


def _node_body(h_ref, d_ref, gi_ref, wp1t, bp1, wp2t, bp2, wpsit, bpsi,
               hout_ref, pool_ref):
    t = pl.program_id(0)
    hn = h_ref[:, :H] + d_ref[0, :, :H] + d_ref[1, :, :H]
    a = jnp.dot(hn, wp1t[...], preferred_element_type=_f32) + bp1[...]
    b = jnp.dot(hn, wp2t[...], preferred_element_type=_f32) + bp2[...]
    zp = _elu(a * b)
    oh = (gi_ref[...] == lax.broadcasted_iota(jnp.int32, (_TN, NG), 1)).astype(_f32)
    contrib = lax.dot_general(oh, zp, (((0,), (0,)), ((), ())),
                              preferred_element_type=_f32)

    @pl.when(t == 0)
    def _():
        pool_ref[...] = jnp.zeros_like(pool_ref)

    pool_ref[...] += contrib
    hv = _elu(jnp.dot(hn, wpsit[...], preferred_element_type=_f32) + bpsi[...])
    hout_ref[...] = jnp.concatenate([hv, jnp.zeros((_TN, HT - H), _f32)], axis=1)


def _tc_node(h, delta, gi2d, wp1t, bp1, wp2t, bp2, wpsit, bpsi):
    full = lambda a, b: pl.BlockSpec((a, b), lambda t: (0, 0))
    return pl.pallas_call(
        _node_body,
        grid=(N // _TN,),
        in_specs=[
            pl.BlockSpec((_TN, HT), lambda t: (t, 0)),
            pl.BlockSpec((_NC, _TN, HT), lambda t: (0, t, 0)),
            pl.BlockSpec((_TN, 1), lambda t: (t, 0)),
            full(H, H), full(1, H),
            full(H, H), full(1, H),
            full(H, H), full(1, H),
        ],
        out_specs=(
            pl.BlockSpec((_TN, HT), lambda t: (t, 0)),
            pl.BlockSpec((NG, H), lambda t: (0, 0)),
        ),
        out_shape=(
            jax.ShapeDtypeStruct((N, HT), _f32),
            jax.ShapeDtypeStruct((NG, H), _f32),
        ),
    )(h, delta, gi2d, wp1t, bp1, wp2t, bp2, wpsit, bpsi)


def _head_body(p0, p1, p2, w1, b1, w2, b2, y_ref):
    p = p0[...] + p1[...] + p2[...]
    y = _elu(jnp.dot(p, w1[...], preferred_element_type=_f32) + b1[...])
    y_ref[...] = _elu(jnp.dot(y, w2[...], preferred_element_type=_f32) + b2[...])


def _tc_head(p0, p1, p2, w1, b1, w2, b2):
    return pl.pallas_call(
        _head_body,
        out_shape=jax.ShapeDtypeStruct((NG, H // 4), _f32),
    )(p0, p1, p2, w1, b1, w2, b2)


def kernel(x, edge_distance, node_counts, combine_sets, plane_wave, W_emb,
           W1v, b1v, W2vg, b2vg, W2v, b2v, Wg, bg, Wm, bm, Wp1, bp1, Wp2,
           bp2, Wpsi, bpsi, Wlr1, blr1, Wlr2, blr2, edge_sources,
           edge_targets, graph_indices):
    pad = jnp.zeros((E_PAD - E,), jnp.int32)
    src2d = jnp.concatenate([edge_sources.astype(jnp.int32), pad]
                            ).reshape(E_PAD // _ROW, _ROW)
    dst2d = jnp.concatenate([edge_targets.astype(jnp.int32), pad]
                            ).reshape(E_PAD // _ROW, _ROW)
    gi2d = graph_indices.astype(jnp.int32).reshape(N, 1)
    rij = edge_distance.reshape(E, 1)
    zeros_tile = jnp.zeros((_ZCH, HT), _f32)

    h = _tc_embed(x, W_emb.T)

    pools = []
    for i in range(3):
        ga1 = jnp.concatenate([Wg[i][:, :H].T, Wm[i][:, :H].T], axis=1)
        ga2 = jnp.concatenate([Wg[i][:, H:2 * H].T, Wm[i][:, H:2 * H].T], axis=1)
        ga3 = jnp.concatenate([Wg[i][:, 2 * H:].T, Wm[i][:, 2 * H:].T], axis=1)
        bgm = jnp.concatenate([bg[i], bm[i]]).reshape(1, 2 * H)

        ni, nj = _sc_gather(h, src2d, dst2d)
        z = _tc_edge(
            ni, nj, rij, combine_sets, plane_wave,
            ga1, ga2, ga3, bgm,
            W1v[i].T, b1v[i].reshape(1, H),
            W2vg[i].T, b2vg[i].reshape(1, H),
            W2v[i].T, b2v[i].reshape(1, H),
        )
        delta = _sc_scatter(z, src2d, zeros_tile)
        h, pool = _tc_node(
            h, delta, gi2d,
            Wp1[i].T, bp1[i].reshape(1, H),
            Wp2[i].T, bp2[i].reshape(1, H),
            Wpsi[i].T, bpsi[i].reshape(1, H),
        )
        pools.append(pool)

    return _tc_head(
        pools[0], pools[1], pools[2],
        Wlr1.T, blr1.reshape(1, H // 2),
        Wlr2.T, blr2.reshape(1, H // 4),
    )


# speedup vs baseline: 3.3363x; 2.2417x over previous
import functools

import jax
import jax.numpy as jnp
from jax import lax
from jax.experimental import pallas as pl
from jax.experimental.pallas import tpu as pltpu
from jax.experimental.pallas import tpu_sc as plsc

N = 10000
E = 320000
D_IN = 128
H = 64
NG = 128
CUTOFF = 8.0

HT = 128
_NC = 2
_NS = 16
_NW = _NC * _NS
_ROW = 128
_GRP = 8
E_PAD = 327680
_ROWS_PER_W = (E_PAD // _ROW) // _NW
_GRPS_PER_W = _ROWS_PER_W // _GRP
_EDGES_PER_W = E_PAD // _NW
N_PAD = 10240
_NPT = N_PAD // _NS
_ZCH = 128

_f32 = jnp.float32


def _sigmoid(v):
    return 1.0 / (1.0 + jnp.exp(-v))


def _elu(v):
    return jnp.where(v > 0, v, jnp.exp(jnp.minimum(v, 0.0)) - 1.0)


_MESH = None


def _mesh():
    global _MESH
    if _MESH is None:
        _MESH = plsc.VectorSubcoreMesh(core_axis_name="c", subcore_axis_name="s")
    return _MESH


def _gather_body(h_hbm, src_hbm, dst_hbm, ni_hbm, nj_hbm,
                 table_sh, idx_v, slot0, slot1, osem0, osem1):
    cid = lax.axis_index("c")
    sid = lax.axis_index("s")
    wid = sid * _NC + cid

    for k in range(_NPT // _ZCH):
        r0 = sid * _NPT + k * _ZCH
        pltpu.sync_copy(h_hbm.at[pl.ds(r0, _ZCH), :], slot0)
        pltpu.sync_copy(slot0, table_sh.at[pl.ds(r0, _ZCH), :])
    plsc.subcore_barrier()

    slots = (slot0, slot1)
    osems = (osem0, osem1)

    def run(ihbm, ohbm, first):
        def group(g, carry):
            row0 = wid * _ROWS_PER_W + g * _GRP
            pltpu.sync_copy(ihbm.at[pl.ds(row0, _GRP), :], idx_v)
            for j in range(_GRP):
                s = j & 1

                def drain(s=s):
                    pltpu.make_async_copy(
                        ohbm.at[pl.ds(0, _ROW), :], slots[s], osems[s]
                    ).wait()

                if first and j < 2:
                    @pl.when(g > 0)
                    def _():
                        drain()
                else:
                    drain()
                pltpu.sync_copy(table_sh.at[idx_v.at[j]], slots[s])
                e0 = wid * _EDGES_PER_W + (g * _GRP + j) * _ROW
                pltpu.async_copy(slots[s], ohbm.at[pl.ds(e0, _ROW), :], osems[s])
            return carry

        lax.fori_loop(0, _GRPS_PER_W, group, 0)

    run(src_hbm, ni_hbm, True)
    run(dst_hbm, nj_hbm, False)
    pltpu.make_async_copy(nj_hbm.at[pl.ds(0, _ROW), :], slot0, osem0).wait()
    pltpu.make_async_copy(nj_hbm.at[pl.ds(0, _ROW), :], slot1, osem1).wait()


def _sc_gather(h, src2d, dst2d):
    f = functools.partial(
        pl.kernel,
        out_type=(
            jax.ShapeDtypeStruct((E_PAD, HT), _f32),
            jax.ShapeDtypeStruct((E_PAD, HT), _f32),
        ),
        mesh=_mesh(),
        scratch_types=[
            pltpu.VMEM_SHARED((N_PAD, HT), _f32),
            pltpu.VMEM((_GRP, _ROW), jnp.int32),
            pltpu.VMEM((_ROW, HT), _f32),
            pltpu.VMEM((_ROW, HT), _f32),
            pltpu.SemaphoreType.DMA,
            pltpu.SemaphoreType.DMA,
        ],
    )(_gather_body)
    return f(h, src2d, dst2d)


def _scatter_body(z_hbm, idx_hbm, zero_hbm, out_hbm, table_sh, idx_v, z_v):
    cid = lax.axis_index("c")
    sid = lax.axis_index("s")
    wid = sid * _NC + cid

    pltpu.sync_copy(zero_hbm, z_v)
    for k in range(_NPT // _ZCH):
        pltpu.sync_copy(z_v, table_sh.at[pl.ds(sid * _NPT + k * _ZCH, _ZCH), :])
    plsc.subcore_barrier()

    def group(g, carry):
        row0 = wid * _ROWS_PER_W + g * _GRP
        pltpu.sync_copy(idx_hbm.at[pl.ds(row0, _GRP), :], idx_v)
        for j in range(_GRP):
            e0 = wid * _EDGES_PER_W + (g * _GRP + j) * _ROW
            pltpu.sync_copy(z_hbm.at[pl.ds(e0, _ROW), :], z_v)
            pltpu.sync_copy(z_v, table_sh.at[idx_v.at[j]], add=True)
        return carry

    lax.fori_loop(0, _GRPS_PER_W, group, 0)
    plsc.subcore_barrier()

    for k in range(_NPT // _ZCH):
        r0 = sid * _NPT + k * _ZCH
        pltpu.sync_copy(table_sh.at[pl.ds(r0, _ZCH), :], z_v)
        pltpu.sync_copy(z_v, out_hbm.at[cid, pl.ds(r0, _ZCH), :])


def _sc_scatter(z, idx2d, zeros_tile):
    f = functools.partial(
        pl.kernel,
        out_type=jax.ShapeDtypeStruct((_NC, N_PAD, HT), _f32),
        mesh=_mesh(),
        scratch_types=[
            pltpu.VMEM_SHARED((N_PAD, HT), _f32),
            pltpu.VMEM((_GRP, _ROW), jnp.int32),
            pltpu.VMEM((_ZCH, HT), _f32),
        ],
    )(_scatter_body)
    return f(z, idx2d, zeros_tile)


_TN = 2048
_TE = 2048


def _embed_body(x_ref, w_ref, h_ref):
    t = pl.program_id(0)
    hv = _sigmoid(jnp.dot(x_ref[...], w_ref[...], preferred_element_type=_f32))
    ids = t * _TN + lax.broadcasted_iota(jnp.int32, (_TN, 1), 0)
    hv = jnp.where(ids < N, hv, 0.0)
    h_ref[...] = jnp.concatenate([hv, jnp.zeros((_TN, HT - H), _f32)], axis=1)


def _tc_embed(x, w_embT):
    return pl.pallas_call(
        _embed_body,
        grid=(N_PAD // _TN,),
        in_specs=[
            pl.BlockSpec((_TN, D_IN), lambda t: (t, 0)),
            pl.BlockSpec((D_IN, H), lambda t: (0, 0)),
        ],
        out_specs=pl.BlockSpec((_TN, HT), lambda t: (t, 0)),
        out_shape=jax.ShapeDtypeStruct((N_PAD, HT), _f32),
    )(x, w_embT)


def _edge_body(ni_ref, nj_ref, r_ref, cs_ref, pw_ref, ga1, ga2, ga3, bgm,
               w1t, b1, wgt, bgv, w2t, b2, z_ref):
    t = pl.program_id(0)
    rv = r_ref[...]
    inv = 1.0 / rv
    ni = ni_ref[:, :H]
    nj = nj_ref[:, :H]
    d = (ni - nj) * inv
    gm = (
        jnp.dot(ni, ga1[...], preferred_element_type=_f32)
        + jnp.dot(nj, ga2[...], preferred_element_type=_f32)
        + jnp.dot(d, ga3[...], preferred_element_type=_f32)
        + bgm[...]
    )
    g = _sigmoid(gm[:, :H])
    m = _elu(gm[:, H:])
    pw = pw_ref[...]
    z1 = jnp.dot(cs_ref[...], w1t[...], preferred_element_type=_f32) + b1[...]
    gt = _sigmoid(jnp.dot(pw, wgt[...], preferred_element_type=_f32) + bgv[...])
    z2 = jnp.dot(pw * gt, w2t[...], preferred_element_type=_f32) + b2[...]
    ids = t * _TE + lax.broadcasted_iota(jnp.int32, (_TE, 1), 0)
    keep = jnp.logical_and(ids < E, rv < CUTOFF)
    zv = jnp.where(keep, g * m * (z1 + z2), 0.0)
    z_ref[...] = jnp.concatenate([zv, jnp.zeros((_TE, HT - H), _f32)], axis=1)


def _tc_edge(ni, nj, rij, cs, pw, ga1, ga2, ga3, bgm, w1t, b1, wgt, bgv, w2t, b2):
    full = lambda a, b: pl.BlockSpec((a, b), lambda t: (0, 0))
    clamp = lambda t: (jnp.minimum(t, E // _TE), 0)
    return pl.pallas_call(
        _edge_body,
        grid=(E_PAD // _TE,),
        in_specs=[
            pl.BlockSpec((_TE, HT), lambda t: (t, 0)),
            pl.BlockSpec((_TE, HT), lambda t: (t, 0)),
            pl.BlockSpec((_TE, 1), clamp),
            pl.BlockSpec((_TE, H), clamp),
            pl.BlockSpec((_TE, H), clamp),
            full(H, 2 * H), full(H, 2 * H), full(H, 2 * H), full(1, 2 * H),
            full(H, H), full(1, H),
            full(H, H), full(1, H),
            full(H, H), full(1, H),
        ],
        out_specs=pl.BlockSpec((_TE, HT), lambda t: (t, 0)),
        out_shape=jax.ShapeDtypeStruct((E_PAD, HT), _f32),
    )(ni, nj, rij, cs, pw, ga1, ga2, ga3, bgm, w1t, b1, wgt, bgv, w2t, b2)


def _node_body(h_ref, d_ref, gi_ref, wp1t, bp1, wp2t, bp2, wpsit, bpsi,
               hout_ref, pool_ref):
    t = pl.program_id(0)
    ids = t * _TN + lax.broadcasted_iota(jnp.int32, (_TN, 1), 0)
    valid = ids < N
    hn = h_ref[:, :H] + d_ref[0, :, :H] + d_ref[1, :, :H]
    a = jnp.dot(hn, wp1t[...], preferred_element_type=_f32) + bp1[...]
    b = jnp.dot(hn, wp2t[...], preferred_element_type=_f32) + bp2[...]
    zp = jnp.where(valid, _elu(a * b), 0.0)
    oh = (gi_ref[...] == lax.broadcasted_iota(jnp.int32, (_TN, NG), 1)).astype(_f32)
    contrib = lax.dot_general(oh, zp, (((0,), (0,)), ((), ())),
                              preferred_element_type=_f32)

    @pl.when(t == 0)
    def _():
        pool_ref[...] = jnp.zeros_like(pool_ref)

    pool_ref[...] += contrib
    hv = jnp.where(
        valid,
        _elu(jnp.dot(hn, wpsit[...], preferred_element_type=_f32) + bpsi[...]),
        0.0,
    )
    hout_ref[...] = jnp.concatenate([hv, jnp.zeros((_TN, HT - H), _f32)], axis=1)


def _tc_node(h, delta, gi2d, wp1t, bp1, wp2t, bp2, wpsit, bpsi):
    full = lambda a, b: pl.BlockSpec((a, b), lambda t: (0, 0))
    return pl.pallas_call(
        _node_body,
        grid=(N_PAD // _TN,),
        in_specs=[
            pl.BlockSpec((_TN, HT), lambda t: (t, 0)),
            pl.BlockSpec((_NC, _TN, HT), lambda t: (0, t, 0)),
            pl.BlockSpec((_TN, 1), lambda t: (t, 0)),
            full(H, H), full(1, H),
            full(H, H), full(1, H),
            full(H, H), full(1, H),
        ],
        out_specs=(
            pl.BlockSpec((_TN, HT), lambda t: (t, 0)),
            pl.BlockSpec((NG, H), lambda t: (0, 0)),
        ),
        out_shape=(
            jax.ShapeDtypeStruct((N_PAD, HT), _f32),
            jax.ShapeDtypeStruct((NG, H), _f32),
        ),
    )(h, delta, gi2d, wp1t, bp1, wp2t, bp2, wpsit, bpsi)


def _head_body(p0, p1, p2, w1, b1, w2, b2, y_ref):
    p = p0[...] + p1[...] + p2[...]
    y = _elu(jnp.dot(p, w1[...], preferred_element_type=_f32) + b1[...])
    y_ref[...] = _elu(jnp.dot(y, w2[...], preferred_element_type=_f32) + b2[...])


def _tc_head(p0, p1, p2, w1, b1, w2, b2):
    return pl.pallas_call(
        _head_body,
        out_shape=jax.ShapeDtypeStruct((NG, H // 4), _f32),
    )(p0, p1, p2, w1, b1, w2, b2)


def kernel(x, edge_distance, node_counts, combine_sets, plane_wave, W_emb,
           W1v, b1v, W2vg, b2vg, W2v, b2v, Wg, bg, Wm, bm, Wp1, bp1, Wp2,
           bp2, Wpsi, bpsi, Wlr1, blr1, Wlr2, blr2, edge_sources,
           edge_targets, graph_indices):
    pad = jnp.zeros((E_PAD - E,), jnp.int32)
    src2d = jnp.concatenate([edge_sources.astype(jnp.int32), pad]
                            ).reshape(E_PAD // _ROW, _ROW)
    dst2d = jnp.concatenate([edge_targets.astype(jnp.int32), pad]
                            ).reshape(E_PAD // _ROW, _ROW)
    gi2d = graph_indices.astype(jnp.int32).reshape(N, 1)
    rij = edge_distance.reshape(E, 1)
    zeros_tile = jnp.zeros((_ZCH, HT), _f32)

    h = _tc_embed(x, W_emb.T)

    pools = []
    for i in range(3):
        ga1 = jnp.concatenate([Wg[i][:, :H].T, Wm[i][:, :H].T], axis=1)
        ga2 = jnp.concatenate([Wg[i][:, H:2 * H].T, Wm[i][:, H:2 * H].T], axis=1)
        ga3 = jnp.concatenate([Wg[i][:, 2 * H:].T, Wm[i][:, 2 * H:].T], axis=1)
        bgm = jnp.concatenate([bg[i], bm[i]]).reshape(1, 2 * H)

        ni, nj = _sc_gather(h, src2d, dst2d)
        z = _tc_edge(
            ni, nj, rij, combine_sets, plane_wave,
            ga1, ga2, ga3, bgm,
            W1v[i].T, b1v[i].reshape(1, H),
            W2vg[i].T, b2vg[i].reshape(1, H),
            W2v[i].T, b2v[i].reshape(1, H),
        )
        delta = _sc_scatter(z, src2d, zeros_tile)
        h, pool = _tc_node(
            h, delta, gi2d,
            Wp1[i].T, bp1[i].reshape(1, H),
            Wp2[i].T, bp2[i].reshape(1, H),
            Wpsi[i].T, bpsi[i].reshape(1, H),
        )
        pools.append(pool)

    return _tc_head(
        pools[0], pools[1], pools[2],
        Wlr1.T, blr1.reshape(1, H // 2),
        Wlr2.T, blr2.reshape(1, H // 4),
    )


# speedup vs baseline: 3.8114x; 1.1424x over previous
import functools

import jax
import jax.numpy as jnp
from jax import lax
from jax.experimental import pallas as pl
from jax.experimental.pallas import tpu as pltpu
from jax.experimental.pallas import tpu_sc as plsc

N = 10000
E = 320000
D_IN = 128
H = 64
NG = 128
CUTOFF = 8.0

HT = 128
_NC = 2
_NS = 16
_NW = _NC * _NS
_ROW = 128
_GRP = 8
E_PAD = 327680
_ROWS_PER_W = (E_PAD // _ROW) // _NW
_GRPS_PER_W = _ROWS_PER_W // _GRP
_EDGES_PER_W = E_PAD // _NW
N_PAD = 10240
_NPT = N_PAD // _NS
_ZCH = 128

_f32 = jnp.float32


def _sigmoid(v):
    return 1.0 / (1.0 + jnp.exp(-v))


def _elu(v):
    return jnp.where(v > 0, v, jnp.exp(jnp.minimum(v, 0.0)) - 1.0)


_MESH = None


def _mesh():
    global _MESH
    if _MESH is None:
        _MESH = plsc.VectorSubcoreMesh(core_axis_name="c", subcore_axis_name="s")
    return _MESH


def _gather_body(h_hbm, src_hbm, dst_hbm, ni_hbm, nj_hbm,
                 table_sh, idx_v, slot0, slot1, osem0, osem1):
    cid = lax.axis_index("c")
    sid = lax.axis_index("s")
    wid = sid * _NC + cid

    for k in range(_NPT // _ZCH):
        r0 = sid * _NPT + k * _ZCH
        pltpu.sync_copy(h_hbm.at[pl.ds(r0, _ZCH), :], slot0)
        pltpu.sync_copy(slot0, table_sh.at[pl.ds(r0, _ZCH), :])
    plsc.subcore_barrier()

    slots = (slot0, slot1)
    osems = (osem0, osem1)

    def run(ihbm, ohbm, first):
        def group(g, carry):
            row0 = wid * _ROWS_PER_W + g * _GRP
            pltpu.sync_copy(ihbm.at[pl.ds(row0, _GRP), :], idx_v)
            for j in range(_GRP):
                s = j & 1

                def drain(s=s):
                    pltpu.make_async_copy(
                        ohbm.at[pl.ds(0, _ROW), :], slots[s], osems[s]
                    ).wait()

                if first and j < 2:
                    @pl.when(g > 0)
                    def _():
                        drain()
                else:
                    drain()
                pltpu.sync_copy(table_sh.at[idx_v.at[j]], slots[s])
                e0 = wid * _EDGES_PER_W + (g * _GRP + j) * _ROW
                pltpu.async_copy(slots[s], ohbm.at[pl.ds(e0, _ROW), :], osems[s])
            return carry

        lax.fori_loop(0, _GRPS_PER_W, group, 0)

    run(src_hbm, ni_hbm, True)
    run(dst_hbm, nj_hbm, False)
    pltpu.make_async_copy(nj_hbm.at[pl.ds(0, _ROW), :], slot0, osem0).wait()
    pltpu.make_async_copy(nj_hbm.at[pl.ds(0, _ROW), :], slot1, osem1).wait()


def _sc_gather(h, src2d, dst2d):
    f = functools.partial(
        pl.kernel,
        out_type=(
            jax.ShapeDtypeStruct((E_PAD, HT), _f32),
            jax.ShapeDtypeStruct((E_PAD, HT), _f32),
        ),
        mesh=_mesh(),
        scratch_types=[
            pltpu.VMEM_SHARED((N_PAD, HT), _f32),
            pltpu.VMEM((_GRP, _ROW), jnp.int32),
            pltpu.VMEM((_ROW, HT), _f32),
            pltpu.VMEM((_ROW, HT), _f32),
            pltpu.SemaphoreType.DMA,
            pltpu.SemaphoreType.DMA,
        ],
    )(_gather_body)
    return f(h, src2d, dst2d)


def _scatter_body(z_hbm, idx_hbm, zero_hbm, out_hbm, table_sh, idx_v,
                  zs0, zs1, zsem0, zsem1):
    cid = lax.axis_index("c")
    sid = lax.axis_index("s")
    wid = sid * _NC + cid

    pltpu.sync_copy(zero_hbm, zs0)
    for k in range(_NPT // _ZCH):
        pltpu.sync_copy(zs0, table_sh.at[pl.ds(sid * _NPT + k * _ZCH, _ZCH), :])
    plsc.subcore_barrier()

    zslots = (zs0, zs1)
    zsems = (zsem0, zsem1)
    pltpu.async_copy(
        z_hbm.at[pl.ds(wid * _EDGES_PER_W, _ROW), :], zs0, zsem0)

    def group(g, carry):
        row0 = wid * _ROWS_PER_W + g * _GRP
        pltpu.sync_copy(idx_hbm.at[pl.ds(row0, _GRP), :], idx_v)
        for j in range(_GRP):
            s = j & 1
            pltpu.make_async_copy(
                z_hbm.at[pl.ds(0, _ROW), :], zslots[s], zsems[s]).wait()
            nxt = g * _GRP + j + 1

            @pl.when(nxt < _ROWS_PER_W)
            def _(nxt=nxt, s=s):
                e0n = wid * _EDGES_PER_W + nxt * _ROW
                pltpu.async_copy(
                    z_hbm.at[pl.ds(e0n, _ROW), :], zslots[1 - s], zsems[1 - s])

            pltpu.sync_copy(zslots[s], table_sh.at[idx_v.at[j]], add=True)
        return carry

    lax.fori_loop(0, _GRPS_PER_W, group, 0)
    plsc.subcore_barrier()

    for k in range(_NPT // _ZCH):
        r0 = sid * _NPT + k * _ZCH
        pltpu.sync_copy(table_sh.at[pl.ds(r0, _ZCH), :], zs0)
        pltpu.sync_copy(zs0, out_hbm.at[cid, pl.ds(r0, _ZCH), :])


def _sc_scatter(z, idx2d, zeros_tile):
    f = functools.partial(
        pl.kernel,
        out_type=jax.ShapeDtypeStruct((_NC, N_PAD, HT), _f32),
        mesh=_mesh(),
        scratch_types=[
            pltpu.VMEM_SHARED((N_PAD, HT), _f32),
            pltpu.VMEM((_GRP, _ROW), jnp.int32),
            pltpu.VMEM((_ROW, HT), _f32),
            pltpu.VMEM((_ROW, HT), _f32),
            pltpu.SemaphoreType.DMA,
            pltpu.SemaphoreType.DMA,
        ],
    )(_scatter_body)
    return f(z, idx2d, zeros_tile)


_TN = 2048
_TE = 4096


def _embed_body(x_ref, w_ref, h_ref):
    t = pl.program_id(0)
    hv = _sigmoid(jnp.dot(x_ref[...], w_ref[...], preferred_element_type=_f32))
    ids = t * _TN + lax.broadcasted_iota(jnp.int32, (_TN, 1), 0)
    hv = jnp.where(ids < N, hv, 0.0)
    h_ref[...] = jnp.concatenate([hv, jnp.zeros((_TN, HT - H), _f32)], axis=1)


def _tc_embed(x, w_embT):
    return pl.pallas_call(
        _embed_body,
        grid=(N_PAD // _TN,),
        in_specs=[
            pl.BlockSpec((_TN, D_IN), lambda t: (t, 0)),
            pl.BlockSpec((D_IN, H), lambda t: (0, 0)),
        ],
        out_specs=pl.BlockSpec((_TN, HT), lambda t: (t, 0)),
        out_shape=jax.ShapeDtypeStruct((N_PAD, HT), _f32),
    )(x, w_embT)


def _edge_body(ni_ref, nj_ref, r_ref, cs_ref, pw_ref, gb1, gb2, bgm,
               w1t, b1, wgt, bgv, w2t, b2, z_ref):
    t = pl.program_id(0)
    rv = r_ref[...]
    inv = 1.0 / rv
    ni = ni_ref[:, :H]
    nj = nj_ref[:, :H]
    p = jnp.dot(ni, gb1[...], preferred_element_type=_f32)
    q = jnp.dot(nj, gb2[...], preferred_element_type=_f32)
    gm = (
        p[:, :2 * H] + q[:, :2 * H]
        + inv * (p[:, 2 * H:] - q[:, 2 * H:])
        + bgm[...]
    )
    g = _sigmoid(gm[:, :H])
    m = _elu(gm[:, H:])
    pw = pw_ref[...]
    z1 = jnp.dot(cs_ref[...], w1t[...], preferred_element_type=_f32) + b1[...]
    gt = _sigmoid(jnp.dot(pw, wgt[...], preferred_element_type=_f32) + bgv[...])
    z2 = jnp.dot(pw * gt, w2t[...], preferred_element_type=_f32) + b2[...]
    ids = t * _TE + lax.broadcasted_iota(jnp.int32, (_TE, 1), 0)
    keep = jnp.logical_and(ids < E, rv < CUTOFF)
    zv = jnp.where(keep, g * m * (z1 + z2), 0.0)
    z_ref[...] = jnp.concatenate([zv, jnp.zeros((_TE, HT - H), _f32)], axis=1)


def _tc_edge(ni, nj, rij, cs, pw, gb1, gb2, bgm, w1t, b1, wgt, bgv, w2t, b2):
    full = lambda a, b: pl.BlockSpec((a, b), lambda t: (0, 0))
    clamp = lambda t: (jnp.minimum(t, E // _TE), 0)
    return pl.pallas_call(
        _edge_body,
        grid=(E_PAD // _TE,),
        in_specs=[
            pl.BlockSpec((_TE, HT), lambda t: (t, 0)),
            pl.BlockSpec((_TE, HT), lambda t: (t, 0)),
            pl.BlockSpec((_TE, 1), clamp),
            pl.BlockSpec((_TE, H), clamp),
            pl.BlockSpec((_TE, H), clamp),
            full(H, 4 * H), full(H, 4 * H), full(1, 2 * H),
            full(H, H), full(1, H),
            full(H, H), full(1, H),
            full(H, H), full(1, H),
        ],
        out_specs=pl.BlockSpec((_TE, HT), lambda t: (t, 0)),
        out_shape=jax.ShapeDtypeStruct((E_PAD, HT), _f32),
    )(ni, nj, rij, cs, pw, gb1, gb2, bgm, w1t, b1, wgt, bgv, w2t, b2)


def _node_body(h_ref, d_ref, gi_ref, wp1t, bp1, wp2t, bp2, wpsit, bpsi,
               hout_ref, pool_ref):
    t = pl.program_id(0)
    ids = t * _TN + lax.broadcasted_iota(jnp.int32, (_TN, 1), 0)
    valid = ids < N
    hn = h_ref[:, :H] + d_ref[0, :, :H] + d_ref[1, :, :H]
    a = jnp.dot(hn, wp1t[...], preferred_element_type=_f32) + bp1[...]
    b = jnp.dot(hn, wp2t[...], preferred_element_type=_f32) + bp2[...]
    zp = jnp.where(valid, _elu(a * b), 0.0)
    oh = (gi_ref[...] == lax.broadcasted_iota(jnp.int32, (_TN, NG), 1)).astype(_f32)
    contrib = lax.dot_general(oh, zp, (((0,), (0,)), ((), ())),
                              preferred_element_type=_f32)

    @pl.when(t == 0)
    def _():
        pool_ref[...] = jnp.zeros_like(pool_ref)

    pool_ref[...] += contrib
    hv = jnp.where(
        valid,
        _elu(jnp.dot(hn, wpsit[...], preferred_element_type=_f32) + bpsi[...]),
        0.0,
    )
    hout_ref[...] = jnp.concatenate([hv, jnp.zeros((_TN, HT - H), _f32)], axis=1)


def _tc_node(h, delta, gi2d, wp1t, bp1, wp2t, bp2, wpsit, bpsi):
    full = lambda a, b: pl.BlockSpec((a, b), lambda t: (0, 0))
    return pl.pallas_call(
        _node_body,
        grid=(N_PAD // _TN,),
        in_specs=[
            pl.BlockSpec((_TN, HT), lambda t: (t, 0)),
            pl.BlockSpec((_NC, _TN, HT), lambda t: (0, t, 0)),
            pl.BlockSpec((_TN, 1), lambda t: (t, 0)),
            full(H, H), full(1, H),
            full(H, H), full(1, H),
            full(H, H), full(1, H),
        ],
        out_specs=(
            pl.BlockSpec((_TN, HT), lambda t: (t, 0)),
            pl.BlockSpec((NG, H), lambda t: (0, 0)),
        ),
        out_shape=(
            jax.ShapeDtypeStruct((N_PAD, HT), _f32),
            jax.ShapeDtypeStruct((NG, H), _f32),
        ),
    )(h, delta, gi2d, wp1t, bp1, wp2t, bp2, wpsit, bpsi)


def _head_body(p0, p1, p2, w1, b1, w2, b2, y_ref):
    p = p0[...] + p1[...] + p2[...]
    y = _elu(jnp.dot(p, w1[...], preferred_element_type=_f32) + b1[...])
    y_ref[...] = _elu(jnp.dot(y, w2[...], preferred_element_type=_f32) + b2[...])


def _tc_head(p0, p1, p2, w1, b1, w2, b2):
    return pl.pallas_call(
        _head_body,
        out_shape=jax.ShapeDtypeStruct((NG, H // 4), _f32),
    )(p0, p1, p2, w1, b1, w2, b2)


def kernel(x, edge_distance, node_counts, combine_sets, plane_wave, W_emb,
           W1v, b1v, W2vg, b2vg, W2v, b2v, Wg, bg, Wm, bm, Wp1, bp1, Wp2,
           bp2, Wpsi, bpsi, Wlr1, blr1, Wlr2, blr2, edge_sources,
           edge_targets, graph_indices):
    pad = jnp.zeros((E_PAD - E,), jnp.int32)
    src2d = jnp.concatenate([edge_sources.astype(jnp.int32), pad]
                            ).reshape(E_PAD // _ROW, _ROW)
    dst2d = jnp.concatenate([edge_targets.astype(jnp.int32), pad]
                            ).reshape(E_PAD // _ROW, _ROW)
    gi2d = graph_indices.astype(jnp.int32).reshape(N, 1)
    rij = edge_distance.reshape(E, 1)
    zeros_tile = jnp.zeros((_ZCH, HT), _f32)

    h = _tc_embed(x, W_emb.T)

    pools = []
    for i in range(3):
        ga1 = jnp.concatenate([Wg[i][:, :H].T, Wm[i][:, :H].T], axis=1)
        ga2 = jnp.concatenate([Wg[i][:, H:2 * H].T, Wm[i][:, H:2 * H].T], axis=1)
        ga3 = jnp.concatenate([Wg[i][:, 2 * H:].T, Wm[i][:, 2 * H:].T], axis=1)
        gb1 = jnp.concatenate([ga1, ga3], axis=1)
        gb2 = jnp.concatenate([ga2, ga3], axis=1)
        bgm = jnp.concatenate([bg[i], bm[i]]).reshape(1, 2 * H)

        ni, nj = _sc_gather(h, src2d, dst2d)
        z = _tc_edge(
            ni, nj, rij, combine_sets, plane_wave,
            gb1, gb2, bgm,
            W1v[i].T, b1v[i].reshape(1, H),
            W2vg[i].T, b2vg[i].reshape(1, H),
            W2v[i].T, b2v[i].reshape(1, H),
        )
        delta = _sc_scatter(z, src2d, zeros_tile)
        h, pool = _tc_node(
            h, delta, gi2d,
            Wp1[i].T, bp1[i].reshape(1, H),
            Wp2[i].T, bp2[i].reshape(1, H),
            Wpsi[i].T, bpsi[i].reshape(1, H),
        )
        pools.append(pool)

    return _tc_head(
        pools[0], pools[1], pools[2],
        Wlr1.T, blr1.reshape(1, H // 2),
        Wlr2.T, blr2.reshape(1, H // 4),
    )
